# Initial kernel scaffold; baseline (speedup 1.0000x reference)
import jax
import jax.numpy as jnp
from jax.experimental import pallas as pl


def kernel(state, conv1_w, conv1_b, conv2_w, conv2_b, w1, b1, w2, b2, w3, b3, w4, b4, w6, b6):
    raise NotImplementedError("write your pallas kernel here")


# speedup vs baseline: 7.5189x; 7.5189x over previous
import jax
import jax.numpy as jnp
from jax.experimental import pallas as pl
from jax.experimental.pallas import tpu as pltpu

_FEAT = 1690
_BB = 128


def _banded(w, h_in):
    O, C, K, _ = w.shape
    h_out = h_in - K + 1
    i = jnp.arange(h_out)
    r = jnp.arange(h_in)
    ki = r[None, :] - i[:, None]
    valid = (ki >= 0) & (ki < K)
    a = w[:, :, jnp.clip(ki, 0, K - 1), :]
    a = jnp.where(valid[None, None, :, :, None], a, 0.0)
    a = a.transpose(0, 2, 4, 1, 3)
    return a.reshape(O * h_out, K * C * h_in)


def _row_sel(C, h):
    rows = jnp.arange(C * (h // 2))
    cols = jnp.arange(C * h)
    c_r, p = rows // (h // 2), rows % (h // 2)
    c_c, q = cols // h, cols % h
    live = c_r[:, None] == c_c[None, :]
    even = (live & (q[None, :] == 2 * p[:, None])).astype(jnp.float32)
    odd = (live & (q[None, :] == 2 * p[:, None] + 1)).astype(jnp.float32)
    return even, odd


def _rotl(x, k):
    return jnp.concatenate([x[:, k:], x[:, :k]], axis=1)


def _fwd_kernel(x_ref, ds_ref, a1_ref, b1r_ref, re1_ref, ro1_ref,
                a2_ref, b2r_ref, re2_ref, ro2_ref,
                w1t_ref, w1bt_ref, b1t_ref, w2t_ref, b2t_ref,
                w3t_ref, b3t_ref, w4t_ref, b4t_ref, w6t_ref, b6t_ref,
                o_ref):
    def mm(a, b):
        return jnp.dot(a, b, preferred_element_type=jnp.float32)

    x = x_ref[0]

    h1 = mm(a1_ref[0:300, :], x)
    for kj in range(1, 5):
        h1 = h1 + mm(a1_ref[kj * 304:kj * 304 + 300, :], _rotl(x, kj * 128))
    h1 = jnp.maximum(h1 + b1r_ref[...], 0.0)

    m1 = jnp.maximum(mm(re1_ref[...], h1), mm(ro1_ref[...], h1))
    q1 = jnp.maximum(m1, _rotl(m1, 128))

    h2 = mm(a2_ref[0:260, :], q1)
    for kj in range(1, 5):
        h2 = h2 + mm(a2_ref[kj * 264:kj * 264 + 260, :], _rotl(q1, kj * 256))
    h2 = jnp.maximum(h2 + b2r_ref[...], 0.0)

    m2 = jnp.maximum(mm(re2_ref[...], h2), mm(ro2_ref[...], h2))
    q2 = jnp.maximum(m2, _rotl(m2, 256))

    q2p = jnp.concatenate([q2, jnp.zeros((6, q2.shape[1]), jnp.float32)], axis=0)
    feat = jnp.concatenate(
        [q2p[:, 512 * j3:512 * j3 + _BB] for j3 in range(13)], axis=0)

    h = mm(w1t_ref[...], feat) + mm(w1bt_ref[...], ds_ref[0]) + b1t_ref[...]
    h = jnp.maximum(h, 0.0)
    h = jnp.maximum(mm(w2t_ref[...], h) + b2t_ref[...], 0.0)
    h = jnp.maximum(mm(w3t_ref[...], h) + b3t_ref[...], 0.0)
    h = jnp.maximum(mm(w4t_ref[...], h) + b4t_ref[...], 0.0)
    logits = mm(w6t_ref[...], h) + b6t_ref[...]

    mx = jnp.max(logits, axis=0, keepdims=True)
    e = jnp.exp(logits - mx)
    den = jnp.sum(e, axis=0, keepdims=True)
    o_ref[0] = e * pl.reciprocal(den, approx=True)


def kernel(state, conv1_w, conv1_b, conv2_w, conv2_b, w1, b1, w2, b2,
           w3, b3, w4, b4, w6, b6):
    B = state.shape[0]
    G = (B + _BB - 1) // _BB
    Bp = G * _BB
    N = w6.shape[1]

    st = state if Bp == B else jnp.pad(state, ((0, Bp - B), (0, 0)))
    ximg = st[:, 3:].reshape(G, _BB, 64, 64).transpose(0, 2, 3, 1)
    ximg = ximg.reshape(G, 64, 64 * _BB)
    dst = st[:, :3].reshape(G, _BB, 3).transpose(0, 2, 1)

    a1 = _banded(conv1_w, 64)
    a1s = jnp.pad(a1.reshape(300, 5, 64).transpose(1, 0, 2),
                  ((0, 0), (0, 4), (0, 0))).reshape(5 * 304, 64)
    a2 = _banded(conv2_w, 30)
    a2s = jnp.pad(a2.reshape(260, 5, 150).transpose(1, 0, 2),
                  ((0, 0), (0, 4), (0, 0))).reshape(5 * 264, 150)
    b1r = jnp.repeat(conv1_b, 60).reshape(300, 1)
    b2r = jnp.repeat(conv2_b, 26).reshape(260, 1)
    re1, ro1 = _row_sel(5, 60)
    re2, ro2 = _row_sel(10, 26)

    w1a = w1[:_FEAT].reshape(10, 13, 13, 512)
    w1a = w1a.transpose(2, 0, 1, 3).reshape(13, 130, 512)
    w1t = jnp.pad(w1a, ((0, 0), (0, 6), (0, 0))).reshape(13 * 136, 512).T
    w1bt = w1[_FEAT:].T

    c = lambda arr: pl.BlockSpec(arr.shape, lambda g: (0,) * arr.ndim)
    consts = (a1s, b1r, re1, ro1, a2s, b2r, re2, ro2,
              w1t, w1bt, b1.reshape(-1, 1), w2.T, b2.reshape(-1, 1),
              w3.T, b3.reshape(-1, 1), w4.T, b4.reshape(-1, 1),
              w6.T, b6.reshape(-1, 1))

    out = pl.pallas_call(
        _fwd_kernel,
        out_shape=jax.ShapeDtypeStruct((G, N, _BB), jnp.float32),
        grid=(G,),
        in_specs=[
            pl.BlockSpec((1, 64, 64 * _BB), lambda g: (g, 0, 0)),
            pl.BlockSpec((1, 3, _BB), lambda g: (g, 0, 0)),
        ] + [c(a) for a in consts],
        out_specs=pl.BlockSpec((1, N, _BB), lambda g: (g, 0, 0)),
        compiler_params=pltpu.CompilerParams(
            dimension_semantics=("parallel",)),
    )(ximg, dst, *consts)

    return out.transpose(0, 2, 1).reshape(Bp, N)[:B]


# speedup vs baseline: 11.6301x; 1.5468x over previous
import jax
import jax.numpy as jnp
from jax.experimental import pallas as pl
from jax.experimental.pallas import tpu as pltpu

_FEAT = 1690
_BB = 128


def _banded(w, h_in):
    O, C, K, _ = w.shape
    h_out = h_in - K + 1
    i = jnp.arange(h_out)
    r = jnp.arange(h_in)
    ki = r[None, :] - i[:, None]
    valid = (ki >= 0) & (ki < K)
    a = w[:, :, jnp.clip(ki, 0, K - 1), :]
    a = jnp.where(valid[None, None, :, :, None], a, 0.0)
    a = a.transpose(0, 2, 4, 1, 3)
    return a.reshape(O * h_out, K * C * h_in)


def _rotl(x, k):
    return jnp.concatenate([x[:, k:], x[:, :k]], axis=1)


def _fwd_kernel(x_ref, ds_ref, a1_ref, b1r_ref, s1_ref,
                a2_ref, b2r_ref, s2_ref,
                w1t_ref, w1bt_ref, b1t_ref, w2t_ref, b2t_ref,
                w3t_ref, b3t_ref, w4t_ref, b4t_ref, w6t_ref, b6t_ref,
                o_ref):
    def mm(a, b):
        return jnp.dot(a, b, preferred_element_type=jnp.float32)

    x = x_ref[0]

    xs1 = jnp.concatenate(
        [x] + [_rotl(x, kj * 128) for kj in range(1, 5)], axis=0)[:, :60 * _BB]
    h1 = jnp.maximum(mm(a1_ref[...], xs1) + b1r_ref[...], 0.0)

    m1 = jnp.maximum(h1, jnp.roll(h1, -1, axis=0))
    m1 = mm(s1_ref[...], m1)
    q1 = jnp.maximum(m1, _rotl(m1, 128)).astype(jnp.bfloat16)

    q1p = jnp.concatenate(
        [q1, jnp.zeros((2, q1.shape[1]), jnp.bfloat16)], axis=0)
    xs2 = jnp.concatenate(
        [q1p] + [_rotl(q1p, kj * 256) for kj in range(1, 5)], axis=0)
    h2 = jnp.maximum(mm(a2_ref[...], xs2) + b2r_ref[...], 0.0)

    m2 = jnp.maximum(h2, jnp.roll(h2, -1, axis=0))
    m2 = mm(s2_ref[...], m2)
    q2 = jnp.maximum(m2, _rotl(m2, 256))

    q2p = jnp.concatenate([q2, jnp.zeros((6, q2.shape[1]), jnp.float32)], axis=0)
    feat = jnp.concatenate(
        [q2p[:, 512 * j3:512 * j3 + _BB] for j3 in range(13)], axis=0)

    h = mm(w1t_ref[...], feat) + mm(w1bt_ref[...], ds_ref[0]) + b1t_ref[...]
    h = jnp.maximum(h, 0.0)
    h = jnp.maximum(mm(w2t_ref[...], h) + b2t_ref[...], 0.0)
    h = jnp.maximum(mm(w3t_ref[...], h) + b3t_ref[...], 0.0)
    h = jnp.maximum(mm(w4t_ref[...], h) + b4t_ref[...], 0.0)
    logits = mm(w6t_ref[...], h) + b6t_ref[...]

    mx = jnp.max(logits, axis=0, keepdims=True)
    e = jnp.exp(logits - mx)
    den = jnp.sum(e, axis=0, keepdims=True)
    o_ref[0] = e * pl.reciprocal(den, approx=True)


def kernel(state, conv1_w, conv1_b, conv2_w, conv2_b, w1, b1, w2, b2,
           w3, b3, w4, b4, w6, b6):
    B = state.shape[0]
    G = (B + _BB - 1) // _BB
    Bp = G * _BB
    N = w6.shape[1]

    st = state if Bp == B else jnp.pad(state, ((0, Bp - B), (0, 0)))
    ximg = st[:, 3:].astype(jnp.bfloat16).reshape(G, _BB, 64, 64)
    ximg = ximg.transpose(0, 2, 3, 1).reshape(G, 64, 64 * _BB)
    dst = st[:, :3].reshape(G, _BB, 3).transpose(0, 2, 1)

    a1s = _banded(conv1_w, 64).astype(jnp.bfloat16)
    a2 = _banded(conv2_w, 30)
    a2s = jnp.pad(a2.reshape(260, 5, 150),
                  ((0, 0), (0, 0), (0, 2))).reshape(260, 760).astype(jnp.bfloat16)
    b1r = jnp.repeat(conv1_b, 60).reshape(300, 1)
    b2r = jnp.repeat(conv2_b, 26).reshape(260, 1)
    s1 = (jnp.arange(300)[None, :] == 2 * jnp.arange(150)[:, None]
          ).astype(jnp.float32)
    s2 = (jnp.arange(260)[None, :] == 2 * jnp.arange(130)[:, None]
          ).astype(jnp.float32)

    w1a = w1[:_FEAT].reshape(10, 13, 13, 512)
    w1a = w1a.transpose(2, 0, 1, 3).reshape(13, 130, 512)
    w1t = jnp.pad(w1a, ((0, 0), (0, 6), (0, 0))).reshape(13 * 136, 512).T
    w1bt = w1[_FEAT:].T

    c = lambda arr: pl.BlockSpec(arr.shape, lambda g: (0,) * arr.ndim)
    consts = (a1s, b1r, s1, a2s, b2r, s2,
              w1t, w1bt, b1.reshape(-1, 1), w2.T, b2.reshape(-1, 1),
              w3.T, b3.reshape(-1, 1), w4.T, b4.reshape(-1, 1),
              w6.T, b6.reshape(-1, 1))

    out = pl.pallas_call(
        _fwd_kernel,
        out_shape=jax.ShapeDtypeStruct((G, N, _BB), jnp.float32),
        grid=(G,),
        in_specs=[
            pl.BlockSpec((1, 64, 64 * _BB), lambda g: (g, 0, 0)),
            pl.BlockSpec((1, 3, _BB), lambda g: (g, 0, 0)),
        ] + [c(a) for a in consts],
        out_specs=pl.BlockSpec((1, N, _BB), lambda g: (g, 0, 0)),
        compiler_params=pltpu.CompilerParams(
            dimension_semantics=("parallel",)),
    )(ximg, dst, *consts)

    return out.transpose(0, 2, 1).reshape(Bp, N)[:B]


# speedup vs baseline: 13.2377x; 1.1382x over previous
import jax
import jax.numpy as jnp
from jax.experimental import pallas as pl
from jax.experimental.pallas import tpu as pltpu

_FEAT = 1690
_BB = 128


def _banded(w, h_in):
    O, C, K, _ = w.shape
    h_out = h_in - K + 1
    i = jnp.arange(h_out)
    r = jnp.arange(h_in)
    ki = r[None, :] - i[:, None]
    valid = (ki >= 0) & (ki < K)
    a = w[:, :, jnp.clip(ki, 0, K - 1), :]
    a = jnp.where(valid[None, None, :, :, None], a, 0.0)
    a = a.transpose(0, 2, 4, 1, 3)
    return a.reshape(O * h_out, K * C * h_in)


def _rotl(x, k):
    return jnp.concatenate([x[:, k:], x[:, :k]], axis=1)


def _fwd_kernel(x_ref, ds_ref, a1_ref, b1r_ref, s1_ref,
                a2_ref, b2r_ref, s2_ref,
                w1t_ref, w1bt_ref, b1t_ref, w2t_ref, b2t_ref,
                w3t_ref, b3t_ref, w4t_ref, b4t_ref, w6t_ref, b6t_ref,
                o_ref):
    def mm(a, b):
        return jnp.dot(a, b, preferred_element_type=jnp.float32)

    x = jnp.transpose(x_ref[0], (1, 0)).reshape(64, 64 * _BB)

    xs1 = jnp.concatenate(
        [x] + [_rotl(x, kj * 128) for kj in range(1, 5)], axis=0)[:, :60 * _BB]
    h1 = jnp.maximum(mm(a1_ref[...], xs1) + b1r_ref[...], 0.0)

    m1 = jnp.maximum(h1, jnp.roll(h1, -1, axis=0))
    m1 = mm(s1_ref[...], m1)
    q1 = jnp.maximum(m1, _rotl(m1, 128)).astype(jnp.bfloat16)

    q1p = jnp.concatenate(
        [q1, jnp.zeros((2, q1.shape[1]), jnp.bfloat16)], axis=0)
    xs2 = jnp.concatenate(
        [q1p] + [_rotl(q1p, kj * 256) for kj in range(1, 5)], axis=0)
    h2 = jnp.maximum(mm(a2_ref[...], xs2) + b2r_ref[...], 0.0)

    m2 = jnp.maximum(h2, jnp.roll(h2, -1, axis=0))
    m2 = mm(s2_ref[...], m2)
    q2 = jnp.maximum(m2, _rotl(m2, 256))

    q2p = jnp.concatenate([q2, jnp.zeros((6, q2.shape[1]), jnp.float32)], axis=0)
    feat = jnp.concatenate(
        [q2p[:, 512 * j3:512 * j3 + _BB] for j3 in range(13)], axis=0)

    h = mm(w1t_ref[...], feat) + mm(w1bt_ref[...], ds_ref[0]) + b1t_ref[...]
    h = jnp.maximum(h, 0.0)
    h = jnp.maximum(mm(w2t_ref[...], h) + b2t_ref[...], 0.0)
    h = jnp.maximum(mm(w3t_ref[...], h) + b3t_ref[...], 0.0)
    h = jnp.maximum(mm(w4t_ref[...], h) + b4t_ref[...], 0.0)
    logits = mm(w6t_ref[...], h) + b6t_ref[...]

    mx = jnp.max(logits, axis=0, keepdims=True)
    e = jnp.exp(logits - mx)
    den = jnp.sum(e, axis=0, keepdims=True)
    o_ref[0] = e * pl.reciprocal(den, approx=True)


def kernel(state, conv1_w, conv1_b, conv2_w, conv2_b, w1, b1, w2, b2,
           w3, b3, w4, b4, w6, b6):
    B = state.shape[0]
    G = (B + _BB - 1) // _BB
    Bp = G * _BB
    N = w6.shape[1]

    st = state if Bp == B else jnp.pad(state, ((0, Bp - B), (0, 0)))
    ximg = st[:, 3:].astype(jnp.bfloat16).reshape(G, _BB, 4096)
    dst = st[:, :3].reshape(G, _BB, 3).transpose(0, 2, 1)

    a1s = _banded(conv1_w, 64).astype(jnp.bfloat16)
    a2 = _banded(conv2_w, 30)
    a2s = jnp.pad(a2.reshape(260, 5, 150),
                  ((0, 0), (0, 0), (0, 2))).reshape(260, 760).astype(jnp.bfloat16)
    b1r = jnp.repeat(conv1_b, 60).reshape(300, 1)
    b2r = jnp.repeat(conv2_b, 26).reshape(260, 1)
    s1 = (jnp.arange(300)[None, :] == 2 * jnp.arange(150)[:, None]
          ).astype(jnp.float32)
    s2 = (jnp.arange(260)[None, :] == 2 * jnp.arange(130)[:, None]
          ).astype(jnp.float32)

    w1a = w1[:_FEAT].reshape(10, 13, 13, 512)
    w1a = w1a.transpose(2, 0, 1, 3).reshape(13, 130, 512)
    w1t = jnp.pad(w1a, ((0, 0), (0, 6), (0, 0))).reshape(13 * 136, 512).T
    w1bt = w1[_FEAT:].T

    c = lambda arr: pl.BlockSpec(arr.shape, lambda g: (0,) * arr.ndim)
    consts = (a1s, b1r, s1, a2s, b2r, s2,
              w1t, w1bt, b1.reshape(-1, 1), w2.T, b2.reshape(-1, 1),
              w3.T, b3.reshape(-1, 1), w4.T, b4.reshape(-1, 1),
              w6.T, b6.reshape(-1, 1))

    out = pl.pallas_call(
        _fwd_kernel,
        out_shape=jax.ShapeDtypeStruct((G, N, _BB), jnp.float32),
        grid=(G,),
        in_specs=[
            pl.BlockSpec((1, _BB, 4096), lambda g: (g, 0, 0)),
            pl.BlockSpec((1, 3, _BB), lambda g: (g, 0, 0)),
        ] + [c(a) for a in consts],
        out_specs=pl.BlockSpec((1, N, _BB), lambda g: (g, 0, 0)),
        compiler_params=pltpu.CompilerParams(
            dimension_semantics=("parallel",)),
    )(ximg, dst, *consts)

    return out.transpose(0, 2, 1).reshape(Bp, N)[:B]
